# baseline (speedup 1.0000x reference)
import functools

import jax
import jax.numpy as jnp
from jax import lax
from jax.experimental import pallas as pl
from jax.experimental.pallas import tpu as pltpu
from jax.experimental.pallas import tpu_sc as plsc

VOCAB = 1000
DIM = 1000
BATCH = 4096
BLOCK = 8
NB = BATCH * BLOCK
NW = 32
B_PER_W = BATCH // NW
CHUNK_B = 8
CHUNK = CHUNK_B * BLOCK
NCHUNK = B_PER_W // CHUNK_B
NBUF = 2


def _sc_gather(idx, table):
    mesh = plsc.VectorSubcoreMesh(core_axis_name="c", subcore_axis_name="s")

    @functools.partial(
        pl.kernel,
        mesh=mesh,
        compiler_params=pltpu.CompilerParams(use_tc_tiling_on_sc=False),
        out_type=jax.ShapeDtypeStruct((BATCH, BLOCK, DIM), jnp.float32),
        scratch_types=[
            pltpu.VMEM((NCHUNK, CHUNK), jnp.int32),
        ]
        + [pltpu.VMEM((CHUNK, DIM), jnp.float32) for _ in range(NBUF)]
        + [pltpu.SemaphoreType.DMA for _ in range(2 * NBUF)],
    )
    def k(idx_hbm, table_hbm, out_hbm, idx_v, *bufs_sems):
        bufs = bufs_sems[:NBUF]
        gsems = bufs_sems[NBUF : 2 * NBUF]
        wsems = bufs_sems[2 * NBUF :]
        wid = lax.axis_index("s") * 2 + lax.axis_index("c")
        pltpu.sync_copy(idx_hbm.at[wid], idx_v)
        bbase = wid * B_PER_W

        def start_gather(g):
            b = g % NBUF
            return pltpu.async_copy(table_hbm.at[idx_v.at[g]], bufs[b], gsems[b])

        def start_write(g):
            b = g % NBUF
            return [
                pltpu.async_copy(
                    bufs[b].at[pl.ds(j * BLOCK, BLOCK)],
                    out_hbm.at[bbase + g * CHUNK_B + j],
                    wsems[b],
                )
                for j in range(CHUNK_B)
            ]

        gathers = [None] * NBUF
        writes = [None] * NBUF
        gathers[0] = start_gather(0)
        for g in range(NCHUNK):
            b = g % NBUF
            gathers[b].wait()
            writes[b] = start_write(g)
            if g + 1 < NCHUNK:
                b2 = (g + 1) % NBUF
                if writes[b2] is not None:
                    for w in writes[b2]:
                        w.wait()
                gathers[b2] = start_gather(g + 1)
        for ws in writes:
            if ws is not None:
                for w in ws:
                    w.wait()

    return k(idx, table)


def kernel(inputs, table):
    idx = inputs.astype(jnp.int32).reshape(NW, NCHUNK, CHUNK)
    return _sc_gather(idx, table)


# speedup vs baseline: 1.0962x; 1.0962x over previous
import functools

import jax
import jax.numpy as jnp
from jax import lax
from jax.experimental import pallas as pl
from jax.experimental.pallas import tpu as pltpu
from jax.experimental.pallas import tpu_sc as plsc

VOCAB = 1000
DIM = 1000
DIMP = 1024
BATCH = 4096
BLOCK = 8
NB = BATCH * BLOCK
NW = 32
B_PER_W = NB // NW
CHUNK = 32
NCHUNK = B_PER_W // CHUNK
NBUF = 2
TC_BB = 128


def _sc_gather(idx, table3):
    mesh = plsc.VectorSubcoreMesh(core_axis_name="c", subcore_axis_name="s")

    @functools.partial(
        pl.kernel,
        mesh=mesh,
        compiler_params=pltpu.CompilerParams(use_tc_tiling_on_sc=False),
        out_type=jax.ShapeDtypeStruct((NB, BLOCK, 128), jnp.float32),
        scratch_types=[
            pltpu.VMEM((NCHUNK, CHUNK), jnp.int32),
        ]
        + [pltpu.VMEM((CHUNK, BLOCK, 128), jnp.float32) for _ in range(NBUF)]
        + [pltpu.SemaphoreType.DMA for _ in range(2 * NBUF)],
    )
    def k(idx_hbm, table_hbm, out_hbm, idx_v, *bufs_sems):
        bufs = bufs_sems[:NBUF]
        gsems = bufs_sems[NBUF : 2 * NBUF]
        wsems = bufs_sems[2 * NBUF :]
        wid = lax.axis_index("s") * 2 + lax.axis_index("c")
        pltpu.sync_copy(idx_hbm.at[wid], idx_v)
        base = wid * B_PER_W

        def start_gather(g):
            b = g % NBUF
            return pltpu.async_copy(table_hbm.at[idx_v.at[g]], bufs[b], gsems[b])

        def start_write(g):
            b = g % NBUF
            return pltpu.async_copy(
                bufs[b], out_hbm.at[pl.ds(base + g * CHUNK, CHUNK)], wsems[b]
            )

        gathers = [None] * NBUF
        writes = [None] * NBUF
        gathers[0] = start_gather(0)
        for g in range(NCHUNK):
            b = g % NBUF
            gathers[b].wait()
            writes[b] = start_write(g)
            if g + 1 < NCHUNK:
                b2 = (g + 1) % NBUF
                if writes[b2] is not None:
                    writes[b2].wait()
                gathers[b2] = start_gather(g + 1)
        for w in writes:
            if w is not None:
                w.wait()

    return k(idx, table3)


def _tc_finish(y):
    def body(y_ref, o_ref):
        v = y_ref[...]
        o_ref[...] = v.reshape(TC_BB, BLOCK, DIMP)[..., :DIM]

    return pl.pallas_call(
        body,
        grid=(BATCH // TC_BB,),
        in_specs=[
            pl.BlockSpec((TC_BB * BLOCK, BLOCK, 128), lambda i: (i, 0, 0)),
        ],
        out_specs=pl.BlockSpec((TC_BB, BLOCK, DIM), lambda i: (i, 0, 0)),
        out_shape=jax.ShapeDtypeStruct((BATCH, BLOCK, DIM), jnp.float32),
    )(y)


def kernel(inputs, table):
    idx = inputs.astype(jnp.int32).reshape(NW, NCHUNK, CHUNK)
    table3 = jnp.pad(table, ((0, 0), (0, DIMP - DIM))).reshape(VOCAB, BLOCK, 128)
    y = _sc_gather(idx, table3)
    return _tc_finish(y)


# speedup vs baseline: 1.4685x; 1.3397x over previous
import functools

import jax
import jax.numpy as jnp
from jax import lax
from jax.experimental import pallas as pl
from jax.experimental.pallas import tpu as pltpu
from jax.experimental.pallas import tpu_sc as plsc

VOCAB = 1000
DIM = 1000
DIMP = 1024
BATCH = 4096
BLOCK = 8
NT = DIMP // 128
NB = BATCH * BLOCK
NW = 32
B_PER_W = NB // NW
CHUNK = 32
NCHUNK = B_PER_W // CHUNK
NBUF = 2
TC_BB = 128


def _sc_gather(idx, table3):
    mesh = plsc.VectorSubcoreMesh(core_axis_name="c", subcore_axis_name="s")

    @functools.partial(
        pl.kernel,
        mesh=mesh,
        compiler_params=pltpu.CompilerParams(use_tc_tiling_on_sc=False),
        out_type=jax.ShapeDtypeStruct((NT, NB, 128), jnp.float32),
        scratch_types=[
            pltpu.VMEM((NCHUNK, CHUNK), jnp.int32),
        ]
        + [pltpu.VMEM((CHUNK, NT, 128), jnp.float32) for _ in range(NBUF)]
        + [pltpu.SemaphoreType.DMA for _ in range(2 * NBUF)],
    )
    def k(idx_hbm, table_hbm, out_hbm, idx_v, *bufs_sems):
        bufs = bufs_sems[:NBUF]
        gsems = bufs_sems[NBUF : 2 * NBUF]
        wsems = bufs_sems[2 * NBUF :]
        wid = lax.axis_index("s") * 2 + lax.axis_index("c")
        pltpu.sync_copy(idx_hbm.at[wid], idx_v)
        base = wid * B_PER_W

        def start_gather(g):
            b = g % NBUF
            return pltpu.async_copy(table_hbm.at[idx_v.at[g]], bufs[b], gsems[b])

        def start_write(g):
            b = g % NBUF
            return [
                pltpu.async_copy(
                    bufs[b].at[:, t],
                    out_hbm.at[t].at[pl.ds(base + g * CHUNK, CHUNK)],
                    wsems[b],
                )
                for t in range(NT)
            ]

        gathers = [None] * NBUF
        writes = [None] * NBUF
        gathers[0] = start_gather(0)
        for g in range(NCHUNK):
            b = g % NBUF
            gathers[b].wait()
            writes[b] = start_write(g)
            if g + 1 < NCHUNK:
                b2 = (g + 1) % NBUF
                if writes[b2] is not None:
                    for w in writes[b2]:
                        w.wait()
                gathers[b2] = start_gather(g + 1)
        for ws in writes:
            if ws is not None:
                for w in ws:
                    w.wait()

    return k(idx, table3)


def _tc_finish(y):
    def body(y_ref, z_ref):
        v = y_ref[...].reshape(BATCH, 128)
        z_ref[...] = jnp.transpose(v, (1, 0)).reshape(1, 128, BATCH)

    return pl.pallas_call(
        body,
        grid=(NT, BLOCK),
        in_specs=[
            pl.BlockSpec((1, BATCH, 128), lambda t, s: (t, s, 0)),
        ],
        out_specs=pl.BlockSpec((1, 128, BATCH), lambda t, s: (s, t, 0)),
        out_shape=jax.ShapeDtypeStruct((BLOCK, DIM, BATCH), jnp.float32),
    )(y)


def kernel(inputs, table):
    idx = inputs.astype(jnp.int32).T.reshape(NW, NCHUNK, CHUNK)
    table3 = jnp.pad(table, ((0, 0), (0, DIMP - DIM))).reshape(VOCAB, NT, 128)
    y = _sc_gather(idx, table3)
    z = _tc_finish(y)
    return jnp.transpose(z, (2, 0, 1))


# speedup vs baseline: 1.6060x; 1.0936x over previous
import functools

import jax
import jax.numpy as jnp
from jax import lax
from jax.experimental import pallas as pl
from jax.experimental.pallas import tpu as pltpu
from jax.experimental.pallas import tpu_sc as plsc

VOCAB = 1000
DIM = 1000
DIMP = 1024
BATCH = 4096
BLOCK = 8
NT = DIMP // 128
NB = BATCH * BLOCK
NW = 32
NSLICE = 4
SPS = BLOCK // NSLICE
RS = SPS * BATCH
B_PER_W = RS // NW
CHUNK = 32
NCHUNK = B_PER_W // CHUNK
NBUF = 2


def _sc_gather(idx, table3):
    mesh = plsc.VectorSubcoreMesh(core_axis_name="c", subcore_axis_name="s")

    @functools.partial(
        pl.kernel,
        mesh=mesh,
        compiler_params=pltpu.CompilerParams(use_tc_tiling_on_sc=False),
        out_type=jax.ShapeDtypeStruct((NT, RS, 128), jnp.float32),
        scratch_types=[
            pltpu.VMEM((NCHUNK, CHUNK), jnp.int32),
        ]
        + [pltpu.VMEM((CHUNK, NT, 128), jnp.float32) for _ in range(NBUF)]
        + [pltpu.SemaphoreType.DMA for _ in range(2 * NBUF)],
    )
    def k(idx_hbm, table_hbm, out_hbm, idx_v, *bufs_sems):
        bufs = bufs_sems[:NBUF]
        gsems = bufs_sems[NBUF : 2 * NBUF]
        wsems = bufs_sems[2 * NBUF :]
        wid = lax.axis_index("s") * 2 + lax.axis_index("c")
        pltpu.sync_copy(idx_hbm.at[wid], idx_v)
        base = wid * B_PER_W

        def start_gather(g):
            b = g % NBUF
            return pltpu.async_copy(table_hbm.at[idx_v.at[g]], bufs[b], gsems[b])

        def start_write(g):
            b = g % NBUF
            return [
                pltpu.async_copy(
                    bufs[b].at[:, t],
                    out_hbm.at[t].at[pl.ds(base + g * CHUNK, CHUNK)],
                    wsems[b],
                )
                for t in range(NT)
            ]

        gathers = [None] * NBUF
        writes = [None] * NBUF
        gathers[0] = start_gather(0)
        for g in range(NCHUNK):
            b = g % NBUF
            gathers[b].wait()
            writes[b] = start_write(g)
            if g + 1 < NCHUNK:
                b2 = (g + 1) % NBUF
                if writes[b2] is not None:
                    for w in writes[b2]:
                        w.wait()
                gathers[b2] = start_gather(g + 1)
        for ws in writes:
            if ws is not None:
                for w in ws:
                    w.wait()

    return k(idx, table3)


def _tc_finish(y, z_prev, slice_idx):
    def body(y_ref, zp_ref, z_ref):
        del zp_ref
        v = y_ref[...].reshape(BATCH, 128)
        z_ref[...] = jnp.transpose(v, (1, 0)).reshape(1, 128, BATCH)

    kwargs = {}
    operands = [y]
    in_specs = [pl.BlockSpec((1, BATCH, 128), lambda t, s: (t, s, 0))]
    if z_prev is None:
        def body0(y_ref, z_ref):
            v = y_ref[...].reshape(BATCH, 128)
            z_ref[...] = jnp.transpose(v, (1, 0)).reshape(1, 128, BATCH)
        fn = body0
    else:
        fn = body
        operands.append(z_prev)
        in_specs.append(pl.BlockSpec(memory_space=pl.ANY))
        kwargs["input_output_aliases"] = {1: 0}

    return pl.pallas_call(
        fn,
        grid=(NT, SPS),
        in_specs=in_specs,
        out_specs=pl.BlockSpec(
            (1, 128, BATCH),
            lambda t, s, _k=slice_idx: (_k * SPS + s, t, 0),
        ),
        out_shape=jax.ShapeDtypeStruct((BLOCK, DIM, BATCH), jnp.float32),
        **kwargs,
    )(*operands)


def kernel(inputs, table):
    idx = inputs.astype(jnp.int32).T.reshape(NSLICE, NW, NCHUNK, CHUNK)
    table3 = jnp.pad(table, ((0, 0), (0, DIMP - DIM))).reshape(VOCAB, NT, 128)
    z = None
    for k in range(NSLICE):
        y = _sc_gather(idx[k], table3)
        z = _tc_finish(y, z, k)
    return jnp.transpose(z, (2, 0, 1))
